# baseline (speedup 1.0000x reference)
import functools
import jax
import jax.numpy as jnp
from jax import lax
from jax.experimental import pallas as pl
from jax.experimental.pallas import tpu as pltpu
from jax.experimental.pallas import tpu_sc as plsc

N = 10000
K = 32
D = 128
DE = 4

NC = 2
NS = 16
L = 16
NW = NC * NS

GROUP = 4
KB = GROUP * K
RC0 = 384
RC1 = 256
RMAX = max(RC0, RC1)
NPAD = NS * (RC0 + RC1)

SLICES = D // L
HALF = SLICES // 2
NT = 10240
NBUF = 4


def _sc_accumulate(nlist_hbm, edges_hbm, nodes_hbm, a_hbm,
                   nlist_v, rows_bufs, edges_bufs, acc_bufs,
                   gsems, ssems):
    sid = lax.axis_index("s")
    cid = lax.axis_index("c")
    base = jnp.where(cid == 0, sid * RC0, NS * RC0 + sid * RC1)
    nrows = jnp.where(cid == 0, RC0, RC1)
    ngroups = nrows // GROUP

    pltpu.sync_copy(nlist_hbm.at[pl.ds(base * K, RMAX * K)], nlist_v)

    def gather(g, rows, edges_v, sem):
        idx = nlist_v.at[pl.ds(g * KB, KB)]
        pltpu.async_copy(edges_hbm.at[pl.ds(base + g * GROUP, GROUP), :],
                         edges_v, sem)
        pltpu.async_copy(nodes_hbm.at[idx], rows, sem)

    def store(g, acc, sem):
        return pltpu.async_copy(
            acc, a_hbm.at[pl.ds(base + g * GROUP, GROUP), :], sem)

    for b in range(NBUF):
        gather(b, rows_bufs[b], edges_bufs[b], gsems[b])

    def gg_body(gg, _):
        for b in range(NBUF):
            rows, edges_v, acc_v = rows_bufs[b], edges_bufs[b], acc_bufs[b]
            g = gg * NBUF + b
            pltpu.make_async_copy(
                edges_hbm.at[pl.ds(base + g * GROUP, GROUP), :], edges_v,
                gsems[b]).wait()
            pltpu.make_async_copy(
                nodes_hbm.at[nlist_v.at[pl.ds(g * KB, KB)]], rows,
                gsems[b]).wait()

            @pl.when(gg > 0)
            def _():
                pltpu.make_async_copy(
                    acc_v, a_hbm.at[pl.ds(base + (g - NBUF) * GROUP, GROUP), :],
                    ssems[b]).wait()

            for nn in range(GROUP):
                for h in range(2):
                    def jj_body(jj, acc):
                        acc = list(acc)
                        ev = edges_v[nn, pl.ds(jj * 16, 16)]
                        for jq in range(4):
                            j = jj * 4 + jq
                            r = []
                            for tb in range(2):
                                w = rows[nn * K + j,
                                         pl.ds((h * 2 + tb) * L, L)]
                                f0 = lax.bitcast_convert_type(
                                    lax.shift_left(w, 16), jnp.float32)
                                f1 = lax.bitcast_convert_type(
                                    jnp.bitwise_and(w, jnp.int32(-65536)),
                                    jnp.float32)
                                r += [f0, f1]
                            for n in range(DE):
                                e = ev[jq * DE + n]
                                for s in range(HALF):
                                    acc[n * HALF + s] = \
                                        acc[n * HALF + s] + e * r[s]
                        return tuple(acc)

                    zero = jnp.zeros((L,), jnp.float32)
                    acc = lax.fori_loop(0, K // 4, jj_body,
                                        (zero,) * (DE * HALF))
                    for n in range(DE):
                        for s in range(HALF):
                            acc_v[nn, pl.ds(n * D + (h * HALF + s) * L, L)] = \
                                acc[n * HALF + s]

            store(g, acc_v, ssems[b])

            @pl.when(g + NBUF < ngroups)
            def _():
                gather(g + NBUF, rows, edges_v, gsems[b])
        return 0

    lax.fori_loop(0, ngroups // NBUF, gg_body, 0)

    for b in range(NBUF):
        g = ngroups - NBUF + b
        pltpu.make_async_copy(
            acc_bufs[b], a_hbm.at[pl.ds(base + g * GROUP, GROUP), :],
            ssems[b]).wait()


def _sc_stage(nodes_bf, nlist_flat, edges_r):
    mesh = plsc.VectorSubcoreMesh(core_axis_name="c", subcore_axis_name="s")
    k = functools.partial(
        pl.kernel,
        out_type=jax.ShapeDtypeStruct((NPAD, DE * D), jnp.float32),
        mesh=mesh,
        compiler_params=pltpu.CompilerParams(use_tc_tiling_on_sc=False),
        scratch_types=[
            pltpu.VMEM((RMAX * K,), jnp.int32),
            [pltpu.VMEM((KB, D // 2), jnp.int32) for _ in range(NBUF)],
            [pltpu.VMEM((GROUP, K * DE), jnp.float32) for _ in range(NBUF)],
            [pltpu.VMEM((GROUP, DE * D), jnp.float32) for _ in range(NBUF)],
            [pltpu.SemaphoreType.DMA for _ in range(NBUF)],
            [pltpu.SemaphoreType.DMA for _ in range(NBUF)],
        ],
    )(_sc_accumulate)
    return k(nlist_flat, edges_r, nodes_bf)


def _mm_body(a_ref, w_ref, o_ref):
    o_ref[...] = jnp.dot(a_ref[...], w_ref[...],
                         preferred_element_type=jnp.float32) * (1.0 / K)


def _tc_matmul(a, wr):
    blk = 512
    grid = NPAD // blk
    return pl.pallas_call(
        _mm_body,
        grid=(grid,),
        in_specs=[
            pl.BlockSpec((blk, DE * D), lambda i: (i, 0)),
            pl.BlockSpec((DE * D, D), lambda i: (0, 0)),
        ],
        out_specs=pl.BlockSpec((blk, D), lambda i: (i, 0)),
        out_shape=jax.ShapeDtypeStruct((N, D), jnp.float32),
    )(a, wr)


@jax.jit
def kernel(nodes, nlist, edges, W):
    nlist_flat = jnp.pad(nlist.astype(jnp.int32).reshape(N * K),
                         (0, (NPAD - N) * K))
    edges_r = jnp.pad(edges.reshape(N, K * DE), ((0, NPAD - N), (0, 0)))
    nodes_p = jnp.pad(nodes, ((0, NT - N), (0, 0)))
    nodes_bf = (nodes_p.reshape(NT, D // 32, 2, L)
                .transpose(0, 1, 3, 2).reshape(NT, D // 2, 2)
                .astype(jnp.bfloat16))
    nodes_i32 = lax.bitcast_convert_type(nodes_bf, jnp.int32)
    a = _sc_stage(nodes_i32, nlist_flat, edges_r)
    wr = jnp.transpose(W, (2, 0, 1)).reshape(DE * D, D)
    return _tc_matmul(a, wr)


# speedup vs baseline: 1.1621x; 1.1621x over previous
import functools
import jax
import jax.numpy as jnp
from jax import lax
from jax.experimental import pallas as pl
from jax.experimental.pallas import tpu as pltpu
from jax.experimental.pallas import tpu_sc as plsc

N = 10000
K = 32
D = 128
DE = 4

NC = 2
NS = 16
L = 16
NW = NC * NS

GROUP = 4
KB = GROUP * K
RC0 = 384
RC1 = 256
RMAX = max(RC0, RC1)
NPAD = NS * (RC0 + RC1)

SLICES = D // L
HALF = SLICES // 2
NT = 10240
NBUF = 4


def _sc_accumulate(nlist_hbm, edges_hbm, nodes_hbm, a_hbm,
                   nlist_v, rows_bufs, edges_bufs, acc_bufs,
                   gsems, ssems):
    sid = lax.axis_index("s")
    cid = lax.axis_index("c")
    base = jnp.where(cid == 0, sid * RC0, NS * RC0 + sid * RC1)
    nrows = jnp.where(cid == 0, RC0, RC1)
    ngroups = nrows // GROUP

    pltpu.sync_copy(nlist_hbm.at[pl.ds(base * K, RMAX * K)], nlist_v)

    def gather(g, rows, edges_v, sem):
        idx = nlist_v.at[pl.ds(g * KB, KB)]
        pltpu.async_copy(edges_hbm.at[pl.ds(base + g * GROUP, GROUP), :],
                         edges_v, sem)
        pltpu.async_copy(nodes_hbm.at[idx], rows, sem)

    def store(g, acc, sem):
        return pltpu.async_copy(
            acc, a_hbm.at[pl.ds(base + g * GROUP, GROUP), :], sem)

    for b in range(NBUF):
        gather(b, rows_bufs[b], edges_bufs[b], gsems[b])

    def gg_body(gg, _):
        for b in range(NBUF):
            rows, edges_v, acc_v = rows_bufs[b], edges_bufs[b], acc_bufs[b]
            g = gg * NBUF + b
            pltpu.make_async_copy(
                edges_hbm.at[pl.ds(base + g * GROUP, GROUP), :], edges_v,
                gsems[b]).wait()
            pltpu.make_async_copy(
                nodes_hbm.at[nlist_v.at[pl.ds(g * KB, KB)]], rows,
                gsems[b]).wait()

            @pl.when(gg > 0)
            def _():
                pltpu.make_async_copy(
                    acc_v, a_hbm.at[pl.ds(base + (g - NBUF) * GROUP, GROUP), :],
                    ssems[b]).wait()

            for nn in range(GROUP):
                for h in range(2):
                    def jj_body(jj, acc):
                        acc = list(acc)
                        ev = edges_v[nn, pl.ds(jj * 8, 16)]
                        for jq in range(2):
                            j = jj * 2 + jq
                            r = []
                            for tb in range(2):
                                w = rows[nn * K + j,
                                         pl.ds((h * 2 + tb) * L, L)]
                                f0 = lax.bitcast_convert_type(
                                    lax.shift_left(w, 16), jnp.float32)
                                f1 = lax.bitcast_convert_type(
                                    jnp.bitwise_and(w, jnp.int32(-65536)),
                                    jnp.float32)
                                r += [f0, f1]
                            for n in range(DE):
                                e = ev[jq * DE + n]
                                for s in range(HALF):
                                    acc[n * HALF + s] = \
                                        acc[n * HALF + s] + e * r[s]
                        return tuple(acc)

                    zero = jnp.zeros((L,), jnp.float32)
                    acc = lax.fori_loop(0, K // 2, jj_body,
                                        (zero,) * (DE * HALF))
                    for n in range(DE):
                        for s in range(HALF):
                            acc_v[nn, pl.ds(n * D + (h * HALF + s) * L, L)] = \
                                acc[n * HALF + s]

            store(g, acc_v, ssems[b])

            @pl.when(g + NBUF < ngroups)
            def _():
                gather(g + NBUF, rows, edges_v, gsems[b])
        return 0

    lax.fori_loop(0, ngroups // NBUF, gg_body, 0)

    for b in range(NBUF):
        g = ngroups - NBUF + b
        pltpu.make_async_copy(
            acc_bufs[b], a_hbm.at[pl.ds(base + g * GROUP, GROUP), :],
            ssems[b]).wait()


def _sc_stage(nodes_bf, nlist_flat, edges_r):
    mesh = plsc.VectorSubcoreMesh(core_axis_name="c", subcore_axis_name="s")
    k = functools.partial(
        pl.kernel,
        out_type=jax.ShapeDtypeStruct((NPAD, DE * D), jnp.float32),
        mesh=mesh,
        compiler_params=pltpu.CompilerParams(use_tc_tiling_on_sc=False),
        scratch_types=[
            pltpu.VMEM((RMAX * K,), jnp.int32),
            [pltpu.VMEM((KB, D // 2), jnp.int32) for _ in range(NBUF)],
            [pltpu.VMEM((GROUP, K * DE), jnp.float32) for _ in range(NBUF)],
            [pltpu.VMEM((GROUP, DE * D), jnp.float32) for _ in range(NBUF)],
            [pltpu.SemaphoreType.DMA for _ in range(NBUF)],
            [pltpu.SemaphoreType.DMA for _ in range(NBUF)],
        ],
    )(_sc_accumulate)
    return k(nlist_flat, edges_r, nodes_bf)


def _mm_body(a_ref, w_ref, o_ref):
    o_ref[...] = jnp.dot(a_ref[...], w_ref[...],
                         preferred_element_type=jnp.float32) * (1.0 / K)


def _tc_matmul(a, wr):
    blk = 512
    grid = NPAD // blk
    return pl.pallas_call(
        _mm_body,
        grid=(grid,),
        in_specs=[
            pl.BlockSpec((blk, DE * D), lambda i: (i, 0)),
            pl.BlockSpec((DE * D, D), lambda i: (0, 0)),
        ],
        out_specs=pl.BlockSpec((blk, D), lambda i: (i, 0)),
        out_shape=jax.ShapeDtypeStruct((N, D), jnp.float32),
    )(a, wr)


@jax.jit
def kernel(nodes, nlist, edges, W):
    nlist_flat = jnp.pad(nlist.astype(jnp.int32).reshape(N * K),
                         (0, (NPAD - N) * K))
    edges_r = jnp.pad(edges.reshape(N, K * DE), ((0, NPAD - N), (0, 0)))
    nodes_p = jnp.pad(nodes, ((0, NT - N), (0, 0)))
    nodes_bf = (nodes_p.reshape(NT, D // 32, 2, L)
                .transpose(0, 1, 3, 2).reshape(NT, D // 2, 2)
                .astype(jnp.bfloat16))
    nodes_i32 = lax.bitcast_convert_type(nodes_bf, jnp.int32)
    a = _sc_stage(nodes_i32, nlist_flat, edges_r)
    wr = jnp.transpose(W, (2, 0, 1)).reshape(DE * D, D)
    return _tc_matmul(a, wr)
